# Initial kernel scaffold; baseline (speedup 1.0000x reference)
import jax
import jax.numpy as jnp
from jax.experimental import pallas as pl


def kernel(x, ensemble_code, table):
    raise NotImplementedError("write your pallas kernel here")


# speedup vs baseline: 2.7059x; 2.7059x over previous
import jax
import jax.numpy as jnp
from jax import lax
from jax.experimental import pallas as pl
from jax.experimental.pallas import tpu as pltpu
from jax.experimental.pallas import tpu_sc as plsc
import numpy as np

EN = 4
NL = 14
FPL = 2
FT = EN * FPL
LOG2H = 19
H = 2 ** LOG2H
BASE_RES = 16
PLS = 1.4472692012786865
N = 65536
NC, NS, LANES = 2, 16, 16
NW = NC * NS
PW = N // NW
CH = LANES
NCHL = PW // CH
P1 = -1640531535
P2 = 805459861
ROWS_PER_CH = 8 * LANES
NTILE = NL * H // 128
TLVL = H // 128
TPWL = TLVL // NW
TBF = 8
NBL = TPWL // TBF
NTB = NL * NBL


def _lv(l):
    scale = BASE_RES * (PLS ** l) - 1.0
    res = int(np.ceil(scale)) + 1
    return scale, res


NDENSE = 5


def _fused(xq, cq, tq, scf, sci, o_t, rows, xv, cv, acc, sclv, resv,
           vin, vout, idx_b, w_b, rows_b, semi, semo, semg0, semg1, semx):
    cid = lax.axis_index("c")
    wid = cid * NS + lax.axis_index("s")
    pltpu.sync_copy(xq.at[:, pl.ds(wid * PW, PW)], xv)
    pltpu.sync_copy(cq.at[:, pl.ds(wid * PW, PW)], cv)
    pltpu.sync_copy(scf, sclv)
    pltpu.sync_copy(sci, resv)
    semg = (semg0, semg1)
    iota = lax.iota(jnp.int32, LANES)
    colconsts = [iota * 0 + f for f in range(FT)]

    def tb_t0(tb):
        lp = tb // NBL
        cb = tb % NBL
        return lp * TLVL + wid * TPWL + cb * TBF

    def fire_in(tb, bv):
        pltpu.async_copy(tq.at[pl.ds(tb_t0(tb), TBF)], vin.at[bv], semi)

    def drain_in(tb, bv):
        pltpu.make_async_copy(tq.at[pl.ds(tb_t0(tb), TBF)], vin.at[bv],
                              semi).wait()

    def fire_out(tb, bv):
        pltpu.async_copy(vout.at[bv],
                         rows.at[pl.ds(tb_t0(tb) * 128, TBF * 128)], semo)

    def drain_out(tb, bv):
        pltpu.make_async_copy(vout.at[bv],
                              rows.at[pl.ds(tb_t0(tb) * 128, TBF * 128)],
                              semo).wait()

    def tbatch(tb):
        bv = tb % 2

        @pl.when(tb + 1 < NTB)
        def _pfi():
            fire_in(tb + 1, 1 - bv)

        @pl.when(tb % NBL >= 2)
        def _dro():
            drain_out(tb - 2, bv)

        drain_in(tb, bv)

        @pl.loop(0, TBF)
        def _tile(k):
            rbase = iota + k * 128
            for g in range(FT):
                ridx = rbase + g * LANES
                for f in range(FT):
                    v = vin[bv, k, f, pl.ds(g * LANES, LANES)]
                    plsc.store_scatter(vout.at[bv], [ridx, colconsts[f]], v)

        fire_out(tb, bv)

    def level_end_drains(lp):
        tb_last = lp * NBL + NBL - 1
        drain_out(tb_last - 1, (tb_last - 1) % 2)
        drain_out(tb_last, tb_last % 2)

    def barrier():
        plsc.subcore_barrier()
        pl.semaphore_signal(semx, 1, core_index=1 - cid)
        pl.semaphore_wait(semx, 1)

    def phase1(l, g, b):
        base = g * CH
        svec = sclv[l, pl.ds(0, LANES)]
        ii, fr = [], []
        for d in range(3):
            p = xv[d, pl.ds(base, CH)] * svec + 0.5
            ip = p.astype(jnp.int32)
            ii.append(ip)
            fr.append(p - ip.astype(jnp.float32))
        w1 = [f * f * (3.0 - 2.0 * f) for f in fr]
        w0 = [1.0 - w for w in w1]
        for c in range(8):
            wc = ((w1[0] if c & 1 else w0[0])
                  * (w1[1] if c & 2 else w0[1])
                  * (w1[2] if c & 4 else w0[2]))
            w_b[b, pl.ds(c * LANES, LANES)] = wc
        lH = iota * 0 + l * H

        @pl.when(l < NDENSE)
        def _dense():
            res = resv[l, pl.ds(0, LANES)]
            rm1 = res - 1
            res2 = res * res
            px = (ii[0], jnp.minimum(ii[0] + 1, rm1))
            py = (ii[1] * res, jnp.minimum(ii[1] + 1, rm1) * res)
            pz = (ii[2] * res2, jnp.minimum(ii[2] + 1, rm1) * res2)
            for c in range(8):
                idx = (px[c & 1] + py[(c >> 1) & 1] + pz[(c >> 2) & 1] + lH)
                idx_b[b, pl.ds(c * LANES, LANES)] = idx

        @pl.when(l >= NDENSE)
        def _hash():
            hy = ii[1] * P1
            hz = ii[2] * P2
            px = (ii[0], ii[0] + 1)
            py = (hy, hy + P1)
            pz = (hz, hz + P2)
            for c in range(8):
                h = px[c & 1] ^ py[(c >> 1) & 1] ^ pz[(c >> 2) & 1]
                idx_b[b, pl.ds(c * LANES, LANES)] = (h & (H - 1)) + lH

    def fire_g(b):
        pltpu.async_copy(rows.at[idx_b.at[b]], rows_b.at[b], semg[b])

    def drain_g(b):
        pltpu.make_async_copy(rows.at[idx_b.at[b]], rows_b.at[b],
                              semg[b]).wait()

    def phase2(l, g, b):
        base = g * CH
        ce = [cv[e, pl.ds(base, CH)] for e in range(EN)]
        rref = rows_b.at[b]
        accv = [jnp.zeros((LANES,), jnp.float32) for _ in range(FPL)]
        for c in range(8):
            wc = w_b[b, pl.ds(c * LANES, LANES)]
            ridx = iota + c * LANES
            cw = [wc * ce[e] for e in range(EN)]
            for e in range(EN):
                for f in range(FPL):
                    v = plsc.load_gather(rref, [ridx, colconsts[e * FPL + f]])
                    accv[f] = accv[f] + cw[e] * v
        for f in range(FPL):
            acc[l * FPL + f, pl.ds(base, CH)] = accv[f]

    fire_in(0, 0)

    @pl.loop(0, NBL)
    def _pro(tb):
        tbatch(tb)

    level_end_drains(0)
    barrier()

    @pl.loop(0, NL)
    def _lvl(l):
        phase1(l, 0, 0)
        fire_g(0)

        @pl.loop(0, NCHL, step=2)
        def _g(g):
            for b in range(2):
                gg = g + b

                @pl.when((l < NL - 1) & (gg % TBF == 0))
                def _tb():
                    tbatch((l + 1) * NBL + gg // TBF)

                @pl.when(gg + 1 < NCHL)
                def _pf():
                    phase1(l, gg + 1, 1 - b)
                    fire_g(1 - b)

                drain_g(b)
                phase2(l, gg, b)

        @pl.when(l < NL - 1)
        def _sync():
            level_end_drains(l + 1)
            barrier()

    pltpu.sync_copy(acc, o_t.at[:, pl.ds(wid * PW, PW)])


def kernel(x, ensemble_code, table):
    xq = x.T
    cq = ensemble_code.T
    tq = (table.transpose(0, 2, 1)
          .reshape(NL, FT, TLVL, 128)
          .transpose(0, 2, 1, 3)
          .reshape(NTILE, FT, 128))
    mesh = plsc.VectorSubcoreMesh(core_axis_name="c", subcore_axis_name="s",
                                  num_cores=NC, num_subcores=NS)
    scales = np.array([_lv(l)[0] for l in range(NL)], np.float32)
    ress = np.array([_lv(l)[1] for l in range(NL)], np.int32)
    scf = jnp.asarray(np.tile(scales[:, None], (1, LANES)))
    sci = jnp.asarray(np.tile(ress[:, None], (1, LANES)))
    scratch = [
        pltpu.VMEM((3, PW), jnp.float32),
        pltpu.VMEM((EN, PW), jnp.float32),
        pltpu.VMEM((NL * FPL, PW), jnp.float32),
        pltpu.VMEM((NL, LANES), jnp.float32),
        pltpu.VMEM((NL, LANES), jnp.int32),
        pltpu.VMEM((2, TBF, FT, 128), jnp.float32),
        pltpu.VMEM((2, TBF * 128, FT), jnp.float32),
        pltpu.VMEM((2, ROWS_PER_CH), jnp.int32),
        pltpu.VMEM((2, 8 * LANES), jnp.float32),
        pltpu.VMEM((2, ROWS_PER_CH, FT), jnp.float32),
        pltpu.SemaphoreType.DMA,
        pltpu.SemaphoreType.DMA,
        pltpu.SemaphoreType.DMA,
        pltpu.SemaphoreType.DMA,
        pltpu.SemaphoreType.REGULAR,
    ]
    f = pl.kernel(_fused,
                  out_type=(jax.ShapeDtypeStruct((NL * FPL, N), jnp.float32),
                            jax.ShapeDtypeStruct((NL * H, FT), jnp.float32)),
                  mesh=mesh, scratch_types=scratch,
                  compiler_params=pltpu.CompilerParams(
                      needs_layout_passes=False,
                      use_tc_tiling_on_sc=False))
    o_t, _ = f(xq, cq, tq, scf, sci)
    return o_t.T


# speedup vs baseline: 4.8811x; 1.8038x over previous
import jax
import jax.numpy as jnp
from jax import lax
from jax.experimental import pallas as pl
from jax.experimental.pallas import tpu as pltpu
from jax.experimental.pallas import tpu_sc as plsc
import numpy as np

EN = 4
NL = 14
FPL = 2
FT = EN * FPL
LOG2H = 19
H = 2 ** LOG2H
BASE_RES = 16
PLS = 1.4472692012786865
N = 65536
NC, NS, LANES = 2, 16, 16
NW = NC * NS
PW = N // NW
CH = LANES
NCH = PW // CH
P1 = -1640531535
P2 = 805459861
ROWS_PER_LVL = 8 * LANES


def _lv(l):
    scale = BASE_RES * (PLS ** l) - 1.0
    res = int(np.ceil(scale)) + 1
    return scale, res


NBUF = 2


def _body(xq, cq, tf, out, xv, cv, idx_b, w_b, rows_b, acc_b,
          sem0, sem1, sem2, sem3):
    wid = lax.axis_index("c") * NS + lax.axis_index("s")
    pltpu.sync_copy(xq.at[:, pl.ds(wid * PW, PW)], xv)
    pltpu.sync_copy(cq.at[:, pl.ds(wid * PW, PW)], cv)
    sems = (sem0, sem1, sem2, sem3)
    iota = lax.iota(jnp.int32, LANES)
    colconsts = [iota * 0 + col for col in range(FT)]
    row0 = wid * PW

    def phase1(g, b):
        base = g * CH
        xs = xv[0, pl.ds(base, CH)]
        ys = xv[1, pl.ds(base, CH)]
        zs = xv[2, pl.ds(base, CH)]
        for l in range(NL):
            scale, res = _lv(l)
            dense = res ** 3 <= H
            ii, fr = [], []
            for v in (xs, ys, zs):
                p = v * scale + 0.5
                ip = p.astype(jnp.int32)
                ii.append(ip)
                fr.append(p - ip.astype(jnp.float32))
            w1 = [f * f * (3.0 - 2.0 * f) for f in fr]
            w0 = [1.0 - w for w in w1]
            for c in range(8):
                wc = ((w1[0] if c & 1 else w0[0])
                      * (w1[1] if c & 2 else w0[1])
                      * (w1[2] if c & 4 else w0[2]))
                w_b[b, pl.ds((l * 8 + c) * LANES, LANES)] = wc
            if dense:
                px = (ii[0], jnp.minimum(ii[0] + 1, res - 1))
                py = (ii[1] * res, jnp.minimum(ii[1] + 1, res - 1) * res)
                pz = (ii[2] * (res * res),
                      jnp.minimum(ii[2] + 1, res - 1) * (res * res))
            else:
                hy = ii[1] * P1
                hz = ii[2] * P2
                px = (ii[0], ii[0] + 1)
                py = (hy, hy + P1)
                pz = (hz, hz + P2)
            for c in range(8):
                a = px[c & 1]
                bpart = py[(c >> 1) & 1]
                cpart = pz[(c >> 2) & 1]
                if dense:
                    idx = a + bpart + cpart + l * H
                else:
                    idx = ((a ^ bpart ^ cpart) & (H - 1)) + l * H
                idx_b[b, l, pl.ds(c * LANES, LANES)] = idx

    def fire(b):
        for l in range(NL):
            pltpu.async_copy(tf.at[idx_b.at[b, l]],
                             rows_b.at[b, pl.ds(l * ROWS_PER_LVL, ROWS_PER_LVL)],
                             sems[b])

    def drain(b):
        for l in range(NL):
            pltpu.make_async_copy(
                tf.at[idx_b.at[b, l]],
                rows_b.at[b, pl.ds(l * ROWS_PER_LVL, ROWS_PER_LVL)],
                sems[b]).wait()

    def phase2(g, b):
        base = g * CH
        ce = [cv[e, pl.ds(base, CH)] for e in range(EN)]
        rows = rows_b.at[b]

        @pl.loop(0, NL)
        def _lvl(l):
            acc = [jnp.zeros((LANES,), jnp.float32) for _ in range(FPL)]
            for c in range(8):
                wc = w_b[b, pl.ds((l * 8 + c) * LANES, LANES)]
                ridx = iota + (l * ROWS_PER_LVL + c * LANES)
                cw = [wc * ce[e] for e in range(EN)]
                for e in range(EN):
                    for f in range(FPL):
                        v = plsc.load_gather(rows, [ridx, colconsts[e * FPL + f]])
                        acc[f] = acc[f] + cw[e] * v
            for f in range(FPL):
                plsc.store_scatter(acc_b, [iota * 0 + (l * FPL + f), iota],
                                   acc[f])

        pltpu.sync_copy(acc_b, out.at[:, pl.ds(row0 + base, CH)])

    for p in range(NBUF - 1):
        phase1(p, p)
        fire(p)

    @pl.loop(0, NCH, step=NBUF)
    def _g(g):
        for b in range(NBUF):
            gg = g + b
            nb = (b + NBUF - 1) % NBUF

            @pl.when(gg + NBUF - 1 < NCH)
            def _prefetch():
                phase1(gg + NBUF - 1, nb)
                fire(nb)

            drain(b)
            phase2(gg, b)


NTILE = NL * H // 128
TB = 16
TPW = NTILE // NW
NBATCH = TPW // TB


def _tx_body(tq, rows, vin, vout, semi, semo):
    wid = lax.axis_index("c") * NS + lax.axis_index("s")
    t0w = wid * TPW
    iota = lax.iota(jnp.int32, LANES)
    colconsts = [iota * 0 + f for f in range(FT)]

    def fire_in(bi, b):
        pltpu.async_copy(tq.at[pl.ds(t0w + bi * TB, TB)], vin.at[b], semi)

    def drain_in(bi, b):
        pltpu.make_async_copy(tq.at[pl.ds(t0w + bi * TB, TB)], vin.at[b],
                              semi).wait()

    def fire_out(bi, b):
        pltpu.async_copy(vout.at[b],
                         rows.at[pl.ds((t0w + bi * TB) * 128, TB * 128)], semo)

    def drain_out(bi, b):
        pltpu.make_async_copy(vout.at[b],
                              rows.at[pl.ds((t0w + bi * TB) * 128, TB * 128)],
                              semo).wait()

    def permute(b):
        @pl.loop(0, TB, unroll=4)
        def _tile(k):
            rbase = iota + k * 128
            for g in range(FT):
                ridx = rbase + g * LANES
                for f in range(FT):
                    v = vin[b, k, f, pl.ds(g * LANES, LANES)]
                    plsc.store_scatter(vout.at[b], [ridx, colconsts[f]], v)

    fire_in(0, 0)

    @pl.loop(0, NBATCH, step=2)
    def _batch(bi):
        for b in range(2):
            bb = bi + b

            @pl.when(bb + 1 < NBATCH)
            def _pf():
                fire_in(bb + 1, 1 - b)

            drain_in(bb, b)

            @pl.when(bb >= 2)
            def _dr():
                drain_out(bb - 2, b)

            permute(b)
            fire_out(bb, b)

    drain_out(NBATCH - 2, 0)
    drain_out(NBATCH - 1, 1)


def _to_row_major(table):
    tq = (table.transpose(0, 2, 1)
          .reshape(NL, FT, NTILE // NL, 128)
          .transpose(0, 2, 1, 3)
          .reshape(NTILE, FT, 128))
    mesh = plsc.VectorSubcoreMesh(core_axis_name="c", subcore_axis_name="s",
                                  num_cores=NC, num_subcores=NS)
    scratch = [
        pltpu.VMEM((2, TB, FT, 128), jnp.float32),
        pltpu.VMEM((2, TB * 128, FT), jnp.float32),
        pltpu.SemaphoreType.DMA,
        pltpu.SemaphoreType.DMA,
    ]
    f = pl.kernel(_tx_body,
                  out_type=jax.ShapeDtypeStruct((NL * H, FT), jnp.float32),
                  mesh=mesh, scratch_types=scratch,
                  compiler_params=pltpu.CompilerParams(
                      needs_layout_passes=False,
                      use_tc_tiling_on_sc=False))
    return f(tq)


def kernel(x, ensemble_code, table):
    xq = x.T
    cq = ensemble_code.T
    tf = _to_row_major(table)
    mesh = plsc.VectorSubcoreMesh(core_axis_name="c", subcore_axis_name="s",
                                  num_cores=NC, num_subcores=NS)
    scratch = [
        pltpu.VMEM((3, PW), jnp.float32),
        pltpu.VMEM((EN, PW), jnp.float32),
        pltpu.VMEM((NBUF, NL, ROWS_PER_LVL), jnp.int32),
        pltpu.VMEM((NBUF, NL * 8 * LANES), jnp.float32),
        pltpu.VMEM((NBUF, NL * ROWS_PER_LVL, FT), jnp.float32),
        pltpu.VMEM((NL * FPL, CH), jnp.float32),
        pltpu.SemaphoreType.DMA,
        pltpu.SemaphoreType.DMA,
        pltpu.SemaphoreType.DMA,
        pltpu.SemaphoreType.DMA,
    ]
    f = pl.kernel(_body,
                  out_type=jax.ShapeDtypeStruct((NL * FPL, N), jnp.float32),
                  mesh=mesh, scratch_types=scratch,
                  compiler_params=pltpu.CompilerParams(
                      needs_layout_passes=False,
                      use_tc_tiling_on_sc=False))
    o_t = f(xq, cq, tf)
    return o_t.T
